# baseline (speedup 1.0000x reference)
import functools

import jax
import jax.numpy as jnp
from jax import lax
from jax.experimental import pallas as pl
from jax.experimental.pallas import tpu as pltpu
from jax.experimental.pallas import tpu_sc as plsc

_FIELD_DIM = 100000
_F = 26
_D = 32
_B = 16384
_N = _B * _F
_NC = 2
_NS = 16
_NW = _NC * _NS
_PER_W = _N // _NW
_CHUNK = 1664
_NCHUNK = _PER_W // _CHUNK
_L = 16
_G = 128
_NGRP = _CHUNK // _G


def _sc_body(x_hbm, table_hbm, out_hbm, idx_v, off_v, rows_v, sem):
    wid = lax.axis_index("s") * _NC + lax.axis_index("c")
    base = wid * _PER_W

    lanes = lax.iota(jnp.int32, _L)

    def init_off(j, _):
        pos = lanes + j * _L
        off_v[pl.ds(j * _L, _L)] = lax.rem(pos, _F) * _FIELD_DIM
        return 0

    lax.fori_loop(0, _CHUNK // _L, init_off, 0)

    def do_chunk(k, _):
        cbase = base + k * _CHUNK
        pltpu.sync_copy(x_hbm.at[pl.ds(cbase, _CHUNK)], idx_v)

        def add_off(j, _):
            sl = pl.ds(j * _L, _L)
            idx_v[sl] = idx_v[sl] + off_v[sl]
            return 0

        lax.fori_loop(0, _CHUNK // _L, add_off, 0)

        copies = []
        for g in range(_NGRP):
            cp = pltpu.make_async_copy(
                table_hbm.at[idx_v.at[pl.ds(g * _G, _G)]],
                rows_v.at[pl.ds(g * _G, _G)],
                sem,
            )
            cp.start()
            copies.append(cp)
        for cp in copies:
            cp.wait()

        pltpu.sync_copy(rows_v, out_hbm.at[pl.ds(cbase, _CHUNK)])
        return 0

    lax.fori_loop(0, _NCHUNK, do_chunk, 0)


_sc_call = functools.partial(
    pl.kernel,
    out_type=jax.ShapeDtypeStruct((_N, _D), jnp.float32),
    scratch_types=[
        pltpu.VMEM((_CHUNK,), jnp.int32),
        pltpu.VMEM((_CHUNK,), jnp.int32),
        pltpu.VMEM((_CHUNK, _D), jnp.float32),
        pltpu.SemaphoreType.DMA,
    ],
    mesh=plsc.VectorSubcoreMesh(core_axis_name="c", subcore_axis_name="s"),
    compiler_params=pltpu.CompilerParams(use_tc_tiling_on_sc=False),
)(_sc_body)


def kernel(x, table):
    out = _sc_call(x.reshape(_N), table)
    return out.reshape(_B, _F, _D)


# speedup vs baseline: 1.1362x; 1.1362x over previous
import functools

import jax
import jax.numpy as jnp
from jax import lax
from jax.experimental import pallas as pl
from jax.experimental.pallas import tpu as pltpu
from jax.experimental.pallas import tpu_sc as plsc

_FIELD_DIM = 100000
_F = 26
_D = 32
_B = 16384
_N = _B * _F
_V = _FIELD_DIM * _F
_NC = 2
_NS = 16
_NW = _NC * _NS
_PER_W = _N // _NW
_CHUNK = 1664
_NCHUNK = _PER_W // _CHUNK
_L = 16
_G = 128
_NGRP = _CHUNK // _G

_TW = 2048
_NBLK = (_V + _TW - 1) // _TW
_VP = _NBLK * _TW


def _repack_body(t_ref, o_ref):
    blk = t_ref[...]
    o_ref[...] = jnp.concatenate(
        [blk[:, 512 * q:512 * (q + 1)].T for q in range(4)], axis=1)


def _repack(table_t):
    return pl.pallas_call(
        _repack_body,
        grid=(_NBLK,),
        in_specs=[pl.BlockSpec((_D, _TW), lambda j: (0, j))],
        out_specs=pl.BlockSpec((_TW // 4, 128), lambda j: (j, 0)),
        out_shape=jax.ShapeDtypeStruct((_VP * _D // 128, 128), jnp.float32),
    )(table_t)


def _sc_body(x_hbm, table_hbm, out_hbm, idx_v, off_v, rows_v, sem):
    wid = lax.axis_index("s") * _NC + lax.axis_index("c")
    base = wid * _PER_W

    lanes = lax.iota(jnp.int32, _L)

    def init_off(j, _):
        pos = lanes + j * _L
        off_v[pl.ds(j * _L, _L)] = lax.rem(pos, _F) * _FIELD_DIM
        return 0

    lax.fori_loop(0, _CHUNK // _L, init_off, 0)

    def do_chunk(k, _):
        cbase = base + k * _CHUNK
        pltpu.sync_copy(x_hbm.at[pl.ds(cbase, _CHUNK)], idx_v)

        def add_off(j, _):
            sl = pl.ds(j * _L, _L)
            v = idx_v[sl] + off_v[sl]
            idx_v[sl] = ((v & ~2047) + ((v & 511) << 2)) + ((v & 2047) >> 9)
            return 0

        lax.fori_loop(0, _CHUNK // _L, add_off, 0)

        copies = []
        for g in range(_NGRP):
            cp = pltpu.make_async_copy(
                table_hbm.at[idx_v.at[pl.ds(g * _G, _G)]],
                rows_v.at[pl.ds(g * _G, _G)],
                sem,
            )
            cp.start()
            copies.append(cp)
        for cp in copies:
            cp.wait()

        pltpu.sync_copy(rows_v, out_hbm.at[pl.ds(cbase, _CHUNK)])
        return 0

    lax.fori_loop(0, _NCHUNK, do_chunk, 0)


_sc_call = functools.partial(
    pl.kernel,
    out_type=jax.ShapeDtypeStruct((_N, _D), jnp.float32),
    scratch_types=[
        pltpu.VMEM((_CHUNK,), jnp.int32),
        pltpu.VMEM((_CHUNK,), jnp.int32),
        pltpu.VMEM((_CHUNK, _D), jnp.float32),
        pltpu.SemaphoreType.DMA,
    ],
    mesh=plsc.VectorSubcoreMesh(core_axis_name="c", subcore_axis_name="s"),
    compiler_params=pltpu.CompilerParams(use_tc_tiling_on_sc=False),
)(_sc_body)


def kernel(x, table):
    packed = _repack(table.T)
    tab_rm = packed.reshape(_VP * _D).reshape(_VP, _D)
    out = _sc_call(x.reshape(_N), tab_rm)
    return out.reshape(_B, _F, _D)


# speedup vs baseline: 1.6377x; 1.4414x over previous
import functools

import jax
import jax.numpy as jnp
from jax import lax
from jax.experimental import pallas as pl
from jax.experimental.pallas import tpu as pltpu
from jax.experimental.pallas import tpu_sc as plsc

_FIELD_DIM = 100000
_F = 26
_D = 32
_B = 16384
_N = _B * _F
_V = _FIELD_DIM * _F
_NC = 2
_NS = 16
_NW = _NC * _NS
_PER_W = _N // _NW
_CHUNK = 1664
_NCHUNK = _PER_W // _CHUNK
_L = 16
_G = 128
_NGRP = _CHUNK // _G

_TW = 8192
_S = _TW // 4
_SLOG = _S.bit_length() - 1
_NBLK = (_V + _TW - 1) // _TW
_VP = _NBLK * _TW


def _repack_body(t_ref, o_ref):
    blk = t_ref[...]
    eye = jnp.eye(_D, dtype=jnp.float32)
    for q in range(4):
        o_ref[:, 32 * q:32 * (q + 1)] = jax.lax.dot_general(
            blk[:, _S * q:_S * (q + 1)], eye, (((0,), (0,)), ((), ())),
            preferred_element_type=jnp.float32)


def _repack(table_t):
    return pl.pallas_call(
        _repack_body,
        grid=(_NBLK,),
        in_specs=[pl.BlockSpec((_D, _TW), lambda j: (0, j))],
        out_specs=pl.BlockSpec((_TW // 4, 128), lambda j: (j, 0)),
        out_shape=jax.ShapeDtypeStruct((_VP * _D // 128, 128), jnp.float32),
        compiler_params=pltpu.CompilerParams(fuse_transposed_lhs_in_matmul=True),
    )(table_t)


def _sc_body(x_hbm, table_hbm, out_hbm, idx_v, off_v, rows_v, sem):
    wid = lax.axis_index("s") * _NC + lax.axis_index("c")
    base = wid * _PER_W

    lanes = lax.iota(jnp.int32, _L)

    def init_off(j, _):
        pos = lanes + j * _L
        off_v[pl.ds(j * _L, _L)] = lax.rem(pos, _F) * _FIELD_DIM
        return 0

    lax.fori_loop(0, _CHUNK // _L, init_off, 0)

    def do_chunk(k, _):
        cbase = base + k * _CHUNK
        pltpu.sync_copy(x_hbm.at[pl.ds(cbase, _CHUNK)], idx_v)

        def add_off(j, _):
            sl = pl.ds(j * _L, _L)
            v = idx_v[sl] + off_v[sl]
            idx_v[sl] = ((v & ~(_TW - 1)) + ((v & (_S - 1)) << 2)) + (
                (v & (_TW - 1)) >> _SLOG)
            return 0

        lax.fori_loop(0, _CHUNK // _L, add_off, 0)

        copies = []
        for g in range(_NGRP):
            cp = pltpu.make_async_copy(
                table_hbm.at[idx_v.at[pl.ds(g * _G, _G)]],
                rows_v.at[pl.ds(g * _G, _G)],
                sem,
            )
            cp.start()
            copies.append(cp)
        for cp in copies:
            cp.wait()

        pltpu.sync_copy(rows_v, out_hbm.at[pl.ds(cbase, _CHUNK)])
        return 0

    lax.fori_loop(0, _NCHUNK, do_chunk, 0)


_sc_call = functools.partial(
    pl.kernel,
    out_type=jax.ShapeDtypeStruct((_N, _D), jnp.float32),
    scratch_types=[
        pltpu.VMEM((_CHUNK,), jnp.int32),
        pltpu.VMEM((_CHUNK,), jnp.int32),
        pltpu.VMEM((_CHUNK, _D), jnp.float32),
        pltpu.SemaphoreType.DMA,
    ],
    mesh=plsc.VectorSubcoreMesh(core_axis_name="c", subcore_axis_name="s"),
    compiler_params=pltpu.CompilerParams(use_tc_tiling_on_sc=False),
)(_sc_body)


def kernel(x, table):
    packed = _repack(table.T)
    tab_rm = packed.reshape(_VP * _D).reshape(_VP, _D)
    out = _sc_call(x.reshape(_N), tab_rm)
    return out.reshape(_B, _F, _D)


# speedup vs baseline: 1.6418x; 1.0025x over previous
import functools

import jax
import jax.numpy as jnp
from jax import lax
from jax.experimental import pallas as pl
from jax.experimental.pallas import tpu as pltpu
from jax.experimental.pallas import tpu_sc as plsc

_FIELD_DIM = 100000
_F = 26
_D = 32
_B = 16384
_N = _B * _F
_V = _FIELD_DIM * _F
_NC = 2
_NS = 16
_NW = _NC * _NS
_PER_W = _N // _NW
_CHUNK = 1664
_NCHUNK = _PER_W // _CHUNK
_L = 16
_G = 128
_NGRP = _CHUNK // _G

_TW = 8192
_S = _TW // 4
_SLOG = _S.bit_length() - 1
_NBLK = (_V + _TW - 1) // _TW
_VP = _NBLK * _TW


def _repack_body(t_ref, o_ref):
    blk = t_ref[...]
    for q in range(4):
        o_ref[:, 32 * q:32 * (q + 1)] = blk[:, _S * q:_S * (q + 1)].T


def _repack(table_t):
    return pl.pallas_call(
        _repack_body,
        grid=(_NBLK,),
        in_specs=[pl.BlockSpec((_D, _TW), lambda j: (0, j))],
        out_specs=pl.BlockSpec((_TW // 4, 128), lambda j: (j, 0)),
        out_shape=jax.ShapeDtypeStruct((_VP * _D // 128, 128), jnp.float32),
        compiler_params=pltpu.CompilerParams(fuse_transposed_lhs_in_matmul=True),
    )(table_t)


def _sc_body(x_hbm, table_hbm, out_hbm, idx_v, off_v, rows_v, sem):
    wid = lax.axis_index("s") * _NC + lax.axis_index("c")
    base = wid * _PER_W

    lanes = lax.iota(jnp.int32, _L)

    def init_off(j, _):
        pos = lanes + j * _L
        off_v[pl.ds(j * _L, _L)] = lax.rem(pos, _F) * _FIELD_DIM
        return 0

    lax.fori_loop(0, _CHUNK // _L, init_off, 0)

    def do_chunk(k, _):
        cbase = base + k * _CHUNK
        pltpu.sync_copy(x_hbm.at[pl.ds(cbase, _CHUNK)], idx_v)

        def add_off(j, _):
            sl = pl.ds(j * _L, _L)
            v = idx_v[sl] + off_v[sl]
            idx_v[sl] = ((v & ~(_TW - 1)) + ((v & (_S - 1)) << 2)) + (
                (v & (_TW - 1)) >> _SLOG)
            return 0

        lax.fori_loop(0, _CHUNK // _L, add_off, 0)

        copies = []
        for g in range(_NGRP):
            cp = pltpu.make_async_copy(
                table_hbm.at[idx_v.at[pl.ds(g * _G, _G)]],
                rows_v.at[pl.ds(g * _G, _G)],
                sem,
            )
            cp.start()
            copies.append(cp)
        for cp in copies:
            cp.wait()

        pltpu.sync_copy(rows_v, out_hbm.at[pl.ds(cbase, _CHUNK)])
        return 0

    lax.fori_loop(0, _NCHUNK, do_chunk, 0)


_sc_call = functools.partial(
    pl.kernel,
    out_type=jax.ShapeDtypeStruct((_N, _D), jnp.float32),
    scratch_types=[
        pltpu.VMEM((_CHUNK,), jnp.int32),
        pltpu.VMEM((_CHUNK,), jnp.int32),
        pltpu.VMEM((_CHUNK, _D), jnp.float32),
        pltpu.SemaphoreType.DMA,
    ],
    mesh=plsc.VectorSubcoreMesh(core_axis_name="c", subcore_axis_name="s"),
    compiler_params=pltpu.CompilerParams(use_tc_tiling_on_sc=False),
)(_sc_body)


def kernel(x, table):
    packed = _repack(table.T)
    tab_rm = packed.reshape(_VP * _D).reshape(_VP, _D)
    out = _sc_call(x.reshape(_N), tab_rm)
    return out.reshape(_B, _F, _D)
